# Initial kernel scaffold; baseline (speedup 1.0000x reference)
import jax
import jax.numpy as jnp
from jax.experimental import pallas as pl


def kernel(x, edge_index, edge_weight, enc1_W, enc1_b, enc2_W, enc2_b, enc3_W, enc3_b, zl_W, zl_b, dec1_W, dec1_b, dec2_W, dec2_b, dec3_W, dec3_b, xbar_W, xbar_b, gnn1_W, gnn2_W, gnn3_W, gnn4_W, gnn5_W, fc1_W, fc1_b, cluster):
    raise NotImplementedError("write your pallas kernel here")


# speedup vs baseline: 1.0220x; 1.0220x over previous
import functools

import jax
import jax.numpy as jnp
from jax.experimental import pallas as pl
from jax.experimental.pallas import tpu as pltpu

N = 10000
SIGMA = 0.3
V = 1.0

ROW_BLK = 2000


def _encoder_body(x_ref, w1, b1, w2, b2, w3, b3, wz, bz,
                  tra1_ref, tra2_ref, tra3_ref, z_ref):
    x = x_ref[...]
    t1 = jax.nn.relu(jnp.dot(x, w1[...], preferred_element_type=jnp.float32) + b1[...])
    tra1_ref[...] = t1
    t2 = jax.nn.relu(jnp.dot(t1, w2[...], preferred_element_type=jnp.float32) + b2[...])
    tra2_ref[...] = t2
    t3 = jax.nn.relu(jnp.dot(t2, w3[...], preferred_element_type=jnp.float32) + b3[...])
    tra3_ref[...] = t3
    z_ref[...] = jnp.dot(t3, wz[...], preferred_element_type=jnp.float32) + bz[...]


def _encoder(x, enc1_W, enc1_b, enc2_W, enc2_b, enc3_W, enc3_b, zl_W, zl_b):
    D_IN, E1 = enc1_W.shape
    E2 = enc2_W.shape[1]
    E3 = enc3_W.shape[1]
    NZ = zl_W.shape[1]
    grid = (N // ROW_BLK,)
    full = lambda shape: pl.BlockSpec(shape, lambda i: (0,) * len(shape))
    row = lambda w: pl.BlockSpec((ROW_BLK, w), lambda i: (i, 0))
    return pl.pallas_call(
        _encoder_body,
        grid=grid,
        in_specs=[
            row(D_IN),
            full((D_IN, E1)), full((E1,)),
            full((E1, E2)), full((E2,)),
            full((E2, E3)), full((E3,)),
            full((E3, NZ)), full((NZ,)),
        ],
        out_specs=[row(E1), row(E2), row(E3), row(NZ)],
        out_shape=[
            jax.ShapeDtypeStruct((N, E1), jnp.float32),
            jax.ShapeDtypeStruct((N, E2), jnp.float32),
            jax.ShapeDtypeStruct((N, E3), jnp.float32),
            jax.ShapeDtypeStruct((N, NZ), jnp.float32),
        ],
    )(x, enc1_W, enc1_b, enc2_W, enc2_b, enc3_W, enc3_b, zl_W, zl_b)


def kernel(x, edge_index, edge_weight, enc1_W, enc1_b, enc2_W, enc2_b, enc3_W, enc3_b,
           zl_W, zl_b, dec1_W, dec1_b, dec2_W, dec2_b, dec3_W, dec3_b, xbar_W, xbar_b,
           gnn1_W, gnn2_W, gnn3_W, gnn4_W, gnn5_W, fc1_W, fc1_b, cluster):
    tra1, tra2, tra3, z = _encoder(
        x, enc1_W, enc1_b, enc2_W, enc2_b, enc3_W, enc3_b, zl_W, zl_b)

    src, dst = edge_index[0], edge_index[1]

    def spmm(sup):
        return jax.ops.segment_sum(sup[src] * edge_weight[:, None], dst, num_segments=N)

    h = jax.nn.relu(spmm(x @ gnn1_W))
    h = jax.nn.relu(spmm(((1 - SIGMA) * h + SIGMA * tra1) @ gnn2_W))
    h = jax.nn.relu(spmm(((1 - SIGMA) * h + SIGMA * tra2) @ gnn3_W))
    h = spmm(((1 - SIGMA) * h + SIGMA * tra3) @ gnn4_W)
    h5 = spmm(((1 - SIGMA) * jax.nn.relu(h) + SIGMA * z) @ gnn5_W)
    predict = jax.nn.softmax(h5, axis=1)

    x_bar = jax.nn.relu(jax.nn.relu(h) @ fc1_W + fc1_b)

    q = 1.0 / (1.0 + jnp.sum((h[:, None, :] - cluster[None, :, :]) ** 2, axis=2) / V)
    q = q ** ((V + 1.0) / 2.0)
    q = q / jnp.sum(q, axis=1, keepdims=True)

    return (x_bar, q, predict, z, h, tra1, tra2, tra3)


# speedup vs baseline: 3.5779x; 3.5011x over previous
import dataclasses
import functools

import jax
import jax.numpy as jnp
from jax import lax
from jax.experimental import pallas as pl
from jax.experimental.pallas import tpu as pltpu
from jax.experimental.pallas import tpu_sc as plsc

N = 10000
E = 320000
SIGMA = 0.3
V = 1.0

ROW_BLK = 2000

NC, NS, L = 2, 16, 16
NW = NC * NS
EPW = E // NW
EB = 80
NBLK = EPW // EB
N_PAD = 10240
RPS = N_PAD // NS
ZR = 128


def _sc_compiler_params():
    cp = pltpu.CompilerParams()
    if "needs_layout_passes" in pltpu.CompilerParams.__dataclass_fields__:
        cp = dataclasses.replace(cp, needs_layout_passes=False)
    if "use_tc_tiling_on_sc" in pltpu.CompilerParams.__dataclass_fields__:
        cp = dataclasses.replace(cp, use_tc_tiling_on_sc=False)
    return cp


def _spmm_sc(sup, src, dst, w):
    width = sup.shape[1]
    mesh = plsc.VectorSubcoreMesh(core_axis_name="c", subcore_axis_name="s")

    @functools.partial(
        pl.kernel,
        compiler_params=_sc_compiler_params(),
        out_type=jax.ShapeDtypeStruct((NC, N_PAD, width), jnp.float32),
        mesh=mesh,
        scratch_types=[
            pltpu.VMEM((1, EB), jnp.int32),
            pltpu.VMEM((1, EB), jnp.int32),
            pltpu.VMEM((EB,), jnp.float32),
            pltpu.VMEM((EB, width), jnp.float32),
            pltpu.VMEM((ZR, width), jnp.float32),
            pltpu.VMEM_SHARED((N_PAD, width), jnp.float32),
        ],
    )
    def k(src_h, dst_h, w_h, sup_h, out_h, srcv, dstv, wv, rows, zrow, acc):
        cid = lax.axis_index("c")
        sid = lax.axis_index("s")
        wid = cid * NS + sid
        base = wid * EPW

        @pl.loop(0, ZR)
        def _(r):
            for c in range(width // L):
                zrow[r, pl.ds(c * L, L)] = jnp.zeros((L,), jnp.float32)

        for j in range(RPS // ZR):
            pltpu.sync_copy(zrow, acc.at[pl.ds(sid * RPS + j * ZR, ZR)])
        plsc.subcore_barrier()

        @pl.loop(0, NBLK)
        def _(b):
            off = base + b * EB
            pltpu.sync_copy(src_h.at[pl.ds(off, EB)], srcv.at[0])
            pltpu.sync_copy(dst_h.at[pl.ds(off, EB)], dstv.at[0])
            pltpu.sync_copy(w_h.at[pl.ds(off, EB)], wv)
            pltpu.sync_copy(sup_h.at[srcv.at[0]], rows)

            @pl.loop(0, EB)
            def _(e):
                ws = plsc.load_gather(wv, [jnp.full((L,), e, jnp.int32)])
                for c in range(width // L):
                    sl = pl.ds(c * L, L)
                    rows[e, sl] = rows[e, sl] * ws

            pltpu.sync_copy(rows, acc.at[dstv.at[0]], add=True)

        plsc.subcore_barrier()
        pltpu.sync_copy(acc.at[pl.ds(sid * RPS, RPS)],
                        out_h.at[cid, pl.ds(sid * RPS, RPS)])

    parts = k(src, dst, w, sup)
    return parts[0, :N] + parts[1, :N]


def _spmm(sup, src, dst, w):
    width = sup.shape[1]
    if width <= 128:
        return _spmm_sc(sup, src, dst, w)
    outs = []
    for c0 in range(0, width, 128):
        outs.append(_spmm_sc(sup[:, c0:c0 + 128], src, dst, w))
    return jnp.concatenate(outs, axis=1)


def _encoder_body(x_ref, w1, b1, w2, b2, w3, b3, wz, bz,
                  tra1_ref, tra2_ref, tra3_ref, z_ref):
    x = x_ref[...]
    t1 = jax.nn.relu(jnp.dot(x, w1[...], preferred_element_type=jnp.float32) + b1[...])
    tra1_ref[...] = t1
    t2 = jax.nn.relu(jnp.dot(t1, w2[...], preferred_element_type=jnp.float32) + b2[...])
    tra2_ref[...] = t2
    t3 = jax.nn.relu(jnp.dot(t2, w3[...], preferred_element_type=jnp.float32) + b3[...])
    tra3_ref[...] = t3
    z_ref[...] = jnp.dot(t3, wz[...], preferred_element_type=jnp.float32) + bz[...]


def _encoder(x, enc1_W, enc1_b, enc2_W, enc2_b, enc3_W, enc3_b, zl_W, zl_b):
    D_IN, E1 = enc1_W.shape
    E2 = enc2_W.shape[1]
    E3 = enc3_W.shape[1]
    NZ = zl_W.shape[1]
    grid = (N // ROW_BLK,)
    full = lambda shape: pl.BlockSpec(shape, lambda i: (0,) * len(shape))
    row = lambda w: pl.BlockSpec((ROW_BLK, w), lambda i: (i, 0))
    return pl.pallas_call(
        _encoder_body,
        grid=grid,
        in_specs=[
            row(D_IN),
            full((D_IN, E1)), full((E1,)),
            full((E1, E2)), full((E2,)),
            full((E2, E3)), full((E3,)),
            full((E3, NZ)), full((NZ,)),
        ],
        out_specs=[row(E1), row(E2), row(E3), row(NZ)],
        out_shape=[
            jax.ShapeDtypeStruct((N, E1), jnp.float32),
            jax.ShapeDtypeStruct((N, E2), jnp.float32),
            jax.ShapeDtypeStruct((N, E3), jnp.float32),
            jax.ShapeDtypeStruct((N, NZ), jnp.float32),
        ],
    )(x, enc1_W, enc1_b, enc2_W, enc2_b, enc3_W, enc3_b, zl_W, zl_b)


def kernel(x, edge_index, edge_weight, enc1_W, enc1_b, enc2_W, enc2_b, enc3_W, enc3_b,
           zl_W, zl_b, dec1_W, dec1_b, dec2_W, dec2_b, dec3_W, dec3_b, xbar_W, xbar_b,
           gnn1_W, gnn2_W, gnn3_W, gnn4_W, gnn5_W, fc1_W, fc1_b, cluster):
    tra1, tra2, tra3, z = _encoder(
        x, enc1_W, enc1_b, enc2_W, enc2_b, enc3_W, enc3_b, zl_W, zl_b)

    src, dst, w = edge_index[0], edge_index[1], edge_weight

    h1 = jax.nn.relu(_spmm(x, src, dst, w) @ gnn1_W)
    u2 = (1 - SIGMA) * h1 + SIGMA * tra1
    h2 = jax.nn.relu(_spmm(u2, src, dst, w) @ gnn2_W)
    u3 = (1 - SIGMA) * h2 + SIGMA * tra2
    h3 = jax.nn.relu(_spmm(u3, src, dst, w) @ gnn3_W)
    u4 = (1 - SIGMA) * h3 + SIGMA * tra3
    h4 = _spmm(u4 @ gnn4_W, src, dst, w)
    u5 = (1 - SIGMA) * jax.nn.relu(h4) + SIGMA * z
    h5 = _spmm(u5 @ gnn5_W, src, dst, w)
    predict = jax.nn.softmax(h5, axis=1)

    x_bar = jax.nn.relu(jax.nn.relu(h4) @ fc1_W + fc1_b)

    q = 1.0 / (1.0 + jnp.sum((h4[:, None, :] - cluster[None, :, :]) ** 2, axis=2) / V)
    q = q ** ((V + 1.0) / 2.0)
    q = q / jnp.sum(q, axis=1, keepdims=True)

    return (x_bar, q, predict, z, h4, tra1, tra2, tra3)


# speedup vs baseline: 4.1667x; 1.1646x over previous
import dataclasses
import functools

import jax
import jax.numpy as jnp
from jax import lax
from jax.experimental import pallas as pl
from jax.experimental.pallas import tpu as pltpu
from jax.experimental.pallas import tpu_sc as plsc

N = 10000
E = 320000
SIGMA = 0.3
V = 1.0

ROW_BLK = 2000

NC, NS, L = 2, 16, 16
NW = NC * NS
EB = 128
NBLKG = E // EB
NBPW = -(-NBLKG // NW)
NPK = (NBPW + 2) * NW
N_PAD = 10240
RPS = N_PAD // NS
ZR = 8


def _sc_compiler_params():
    cp = pltpu.CompilerParams()
    if "needs_layout_passes" in pltpu.CompilerParams.__dataclass_fields__:
        cp = dataclasses.replace(cp, needs_layout_passes=False)
    if "use_tc_tiling_on_sc" in pltpu.CompilerParams.__dataclass_fields__:
        cp = dataclasses.replace(cp, use_tc_tiling_on_sc=False)
    return cp


def _pack_edges(src, dst, w):
    pad = NPK * EB - E
    srcp = jnp.concatenate([src, jnp.zeros((pad,), jnp.int32)])
    dstp = jnp.concatenate([dst, jnp.zeros((pad,), jnp.int32)])
    wp = jnp.concatenate([w, jnp.zeros((pad,), jnp.float32)])
    pk = jnp.stack([srcp, dstp, lax.bitcast_convert_type(wp, jnp.int32)])
    return pk.reshape(3, NPK, EB).transpose(1, 0, 2)


def _spmm_sc(sup, pk):
    width = sup.shape[1]
    mesh = plsc.VectorSubcoreMesh(core_axis_name="c", subcore_axis_name="s")

    @functools.partial(
        pl.kernel,
        compiler_params=_sc_compiler_params(),
        out_type=jax.ShapeDtypeStruct((NC, N_PAD, width), jnp.float32),
        mesh=mesh,
        scratch_types=[
            pltpu.VMEM((3, EB), jnp.int32),
            pltpu.VMEM((3, EB), jnp.int32),
            pltpu.VMEM((EB, width), jnp.float32),
            pltpu.VMEM((EB, width), jnp.float32),
            pltpu.VMEM((ZR, width), jnp.float32),
            pltpu.VMEM_SHARED((N_PAD, width), jnp.float32),
            pltpu.SemaphoreType.DMA,
            pltpu.SemaphoreType.DMA,
            pltpu.SemaphoreType.DMA,
            pltpu.SemaphoreType.DMA,
        ],
    )
    def k(pk_h, sup_h, out_h, pk0, pk1, rows0, rows1, zrow, acc,
          si0, si1, sg0, sg1):
        cid = lax.axis_index("c")
        sid = lax.axis_index("s")
        wid = cid * NS + sid
        pkv = (pk0, pk1)
        rows = (rows0, rows1)
        sem_i = (si0, si1)
        sem_g = (sg0, sg1)

        @pl.loop(0, ZR)
        def _(r):
            for c in range(width // L):
                zrow[r, pl.ds(c * L, L)] = jnp.zeros((L,), jnp.float32)

        for j in range(RPS // ZR):
            pltpu.sync_copy(zrow, acc.at[pl.ds(sid * RPS + j * ZR, ZR)])
        plsc.subcore_barrier()

        def g_of(b):
            return b * NW + wid

        def issue_idx(b, s):
            pltpu.async_copy(pk_h.at[g_of(b)], pkv[s], sem_i[s])

        def wait_idx(s):
            pltpu.make_async_copy(pk_h.at[0], pkv[s], sem_i[s]).wait()

        def issue_gather(s):
            pltpu.async_copy(sup_h.at[pkv[s].at[0]], rows[s], sem_g[s])

        def wait_gather(s):
            pltpu.make_async_copy(sup_h.at[pkv[s].at[0]], rows[s],
                                  sem_g[s]).wait()

        def scale(s):
            @plsc.parallel_loop(0, EB, unroll=4)
            def _(e):
                wi = plsc.load_gather(pkv[s].at[2],
                                      [jnp.full((L,), e, jnp.int32)])
                ws = plsc.bitcast(wi, jnp.float32)
                for c in range(width // L):
                    sl = pl.ds(c * L, L)
                    rows[s][e, sl] = rows[s][e, sl] * ws

        def scatter(s):
            pltpu.sync_copy(rows[s], acc.at[pkv[s].at[1]], add=True)

        def body(b, s, guard):
            wait_gather(s)
            wait_idx(1 - s)
            issue_gather(1 - s)
            scale(s)
            if guard:
                @pl.when(g_of(b) < NBLKG)
                def _():
                    scatter(s)
            else:
                scatter(s)
            issue_idx(b + 2, s)

        issue_idx(0, 0)
        issue_idx(1, 1)
        wait_idx(0)
        issue_gather(0)

        @pl.loop(0, NBPW - 1, step=2)
        def _(b0):
            body(b0, 0, guard=False)
            body(b0 + 1, 1, guard=False)

        body(NBPW - 1, 0, guard=True)

        wait_gather(1)
        wait_idx(0)

        plsc.subcore_barrier()
        pltpu.sync_copy(acc.at[pl.ds(sid * RPS, RPS)],
                        out_h.at[cid, pl.ds(sid * RPS, RPS)])

    parts = k(pk, sup)
    return parts[0, :N] + parts[1, :N]


def _spmm(sup, pk):
    width = sup.shape[1]
    if width <= 128:
        return _spmm_sc(sup, pk)
    outs = []
    for c0 in range(0, width, 128):
        outs.append(_spmm_sc(sup[:, c0:c0 + 128], pk))
    return jnp.concatenate(outs, axis=1)


def _encoder_body(x_ref, w1, b1, w2, b2, w3, b3, wz, bz,
                  tra1_ref, tra2_ref, tra3_ref, z_ref):
    x = x_ref[...]
    t1 = jax.nn.relu(jnp.dot(x, w1[...], preferred_element_type=jnp.float32) + b1[...])
    tra1_ref[...] = t1
    t2 = jax.nn.relu(jnp.dot(t1, w2[...], preferred_element_type=jnp.float32) + b2[...])
    tra2_ref[...] = t2
    t3 = jax.nn.relu(jnp.dot(t2, w3[...], preferred_element_type=jnp.float32) + b3[...])
    tra3_ref[...] = t3
    z_ref[...] = jnp.dot(t3, wz[...], preferred_element_type=jnp.float32) + bz[...]


def _encoder(x, enc1_W, enc1_b, enc2_W, enc2_b, enc3_W, enc3_b, zl_W, zl_b):
    D_IN, E1 = enc1_W.shape
    E2 = enc2_W.shape[1]
    E3 = enc3_W.shape[1]
    NZ = zl_W.shape[1]
    grid = (N // ROW_BLK,)
    full = lambda shape: pl.BlockSpec(shape, lambda i: (0,) * len(shape))
    row = lambda w: pl.BlockSpec((ROW_BLK, w), lambda i: (i, 0))
    return pl.pallas_call(
        _encoder_body,
        grid=grid,
        in_specs=[
            row(D_IN),
            full((D_IN, E1)), full((E1,)),
            full((E1, E2)), full((E2,)),
            full((E2, E3)), full((E3,)),
            full((E3, NZ)), full((NZ,)),
        ],
        out_specs=[row(E1), row(E2), row(E3), row(NZ)],
        out_shape=[
            jax.ShapeDtypeStruct((N, E1), jnp.float32),
            jax.ShapeDtypeStruct((N, E2), jnp.float32),
            jax.ShapeDtypeStruct((N, E3), jnp.float32),
            jax.ShapeDtypeStruct((N, NZ), jnp.float32),
        ],
    )(x, enc1_W, enc1_b, enc2_W, enc2_b, enc3_W, enc3_b, zl_W, zl_b)


def kernel(x, edge_index, edge_weight, enc1_W, enc1_b, enc2_W, enc2_b, enc3_W, enc3_b,
           zl_W, zl_b, dec1_W, dec1_b, dec2_W, dec2_b, dec3_W, dec3_b, xbar_W, xbar_b,
           gnn1_W, gnn2_W, gnn3_W, gnn4_W, gnn5_W, fc1_W, fc1_b, cluster):
    tra1, tra2, tra3, z = _encoder(
        x, enc1_W, enc1_b, enc2_W, enc2_b, enc3_W, enc3_b, zl_W, zl_b)

    pk = _pack_edges(edge_index[0], edge_index[1], edge_weight)

    h1 = jax.nn.relu(_spmm(x, pk) @ gnn1_W)
    u2 = (1 - SIGMA) * h1 + SIGMA * tra1
    h2 = jax.nn.relu(_spmm(u2, pk) @ gnn2_W)
    u3 = (1 - SIGMA) * h2 + SIGMA * tra2
    h3 = jax.nn.relu(_spmm(u3, pk) @ gnn3_W)
    u4 = (1 - SIGMA) * h3 + SIGMA * tra3
    h4 = _spmm(u4 @ gnn4_W, pk)
    u5 = (1 - SIGMA) * jax.nn.relu(h4) + SIGMA * z
    h5 = _spmm(u5 @ gnn5_W, pk)
    predict = jax.nn.softmax(h5, axis=1)

    x_bar = jax.nn.relu(jax.nn.relu(h4) @ fc1_W + fc1_b)

    q = 1.0 / (1.0 + jnp.sum((h4[:, None, :] - cluster[None, :, :]) ** 2, axis=2) / V)
    q = q ** ((V + 1.0) / 2.0)
    q = q / jnp.sum(q, axis=1, keepdims=True)

    return (x_bar, q, predict, z, h4, tra1, tra2, tra3)
